# Initial kernel scaffold; baseline (speedup 1.0000x reference)
import jax
import jax.numpy as jnp
from jax.experimental import pallas as pl


def kernel(xv, adj_pos, adj_neg):
    raise NotImplementedError("write your pallas kernel here")


# speedup vs baseline: 91.7610x; 91.7610x over previous
import dataclasses
import functools

import jax
import jax.numpy as jnp
from jax import lax
from jax.experimental import pallas as pl
from jax.experimental.pallas import tpu as pltpu
from jax.experimental.pallas import tpu_sc as plsc

_P = 5.0
_A = 10.0

_NC = 2
_NS = 16
_LANE = 128
_WROWS = 16
_W = _WROWS * _LANE


def _sc_segment_sums(x, c2, v2, windows_per_tile, n_pad):
    v_nodes = x.shape[0]
    bins_per_tile = n_pad // _NS
    mesh = plsc.VectorSubcoreMesh(core_axis_name="c", subcore_axis_name="s")
    cp = pltpu.CompilerParams()
    if "needs_layout_passes" in pltpu.CompilerParams.__dataclass_fields__:
        cp = dataclasses.replace(cp, needs_layout_passes=False)

    @functools.partial(
        pl.kernel,
        out_type=(
            jax.ShapeDtypeStruct((_NC * n_pad,), jnp.float32),
            jax.ShapeDtypeStruct((_NC * n_pad,), jnp.float32),
        ),
        mesh=mesh,
        compiler_params=cp,
        scratch_types=[
            pltpu.VMEM((v_nodes,), jnp.float32),
            pltpu.VMEM((_WROWS, _LANE), jnp.int32),
            pltpu.VMEM((_WROWS, _LANE), jnp.int32),
            pltpu.VMEM((_WROWS, _LANE), jnp.float32),
            pltpu.VMEM((_WROWS, _LANE), jnp.float32),
            pltpu.VMEM((bins_per_tile,), jnp.float32),
            pltpu.VMEM_SHARED((n_pad,), jnp.float32),
            pltpu.VMEM_SHARED((n_pad,), jnp.float32),
        ],
    )
    def k(x_hbm, c_hbm, v_hbm, num_out, den_out,
          x_v, cidx, vidx, nbuf, ebuf, zbuf, num_sh, den_sh):
        c = lax.axis_index("c")
        s = lax.axis_index("s")

        @pl.loop(0, bins_per_tile, step=16)
        def _(i):
            zbuf[pl.ds(i, 16)] = jnp.zeros((16,), jnp.float32)

        pltpu.sync_copy(zbuf, num_sh.at[pl.ds(s * bins_per_tile, bins_per_tile)])
        pltpu.sync_copy(zbuf, den_sh.at[pl.ds(s * bins_per_tile, bins_per_tile)])

        pltpu.sync_copy(x_hbm, x_v)
        plsc.subcore_barrier()

        def process(is_neg):
            phase = jnp.where(is_neg, 1, 0)

            @pl.loop(0, windows_per_tile)
            def _(w):
                row0 = (s * windows_per_tile + w) * _WROWS
                pltpu.sync_copy(c_hbm.at[phase, pl.ds(row0, _WROWS)], cidx)
                pltpu.sync_copy(v_hbm.at[phase, pl.ds(row0, _WROWS)], vidx)

                @pl.loop(0, _WROWS)
                def _(r):
                    @pl.loop(0, _LANE, step=16)
                    def _(i):
                        vi = vidx[r, pl.ds(i, 16)]
                        xg = plsc.load_gather(x_v, [vi])
                        lit = jnp.where(is_neg, 1.0 - xg, xg)
                        e = jnp.exp(lit * _P)
                        nbuf[r, pl.ds(i, 16)] = lit * e
                        ebuf[r, pl.ds(i, 16)] = e

                @pl.loop(0, _WROWS)
                def _(r):
                    pltpu.sync_copy(nbuf.at[r], num_sh.at[cidx.at[r]], add=True)
                    pltpu.sync_copy(ebuf.at[r], den_sh.at[cidx.at[r]], add=True)

        process(c == 1)

        plsc.subcore_barrier()
        base = c * n_pad + s * bins_per_tile
        pltpu.sync_copy(num_sh.at[pl.ds(s * bins_per_tile, bins_per_tile)], zbuf)
        pltpu.sync_copy(zbuf, num_out.at[pl.ds(base, bins_per_tile)])
        pltpu.sync_copy(den_sh.at[pl.ds(s * bins_per_tile, bins_per_tile)], zbuf)
        pltpu.sync_copy(zbuf, den_out.at[pl.ds(base, bins_per_tile)])

    return k(x, c2, v2)


def _tc_loss(num2, den2, num_clauses):
    def body(n_ref, d_ref, o_ref):
        n = n_ref[0:1, :] + n_ref[1:2, :]
        d = d_ref[0:1, :] + d_ref[1:2, :]
        r = n / d
        sm = 1.0 / (1.0 + jnp.exp(_A * (0.5 - r)))
        col = lax.broadcasted_iota(jnp.int32, sm.shape, 1)
        term = jnp.where(col < num_clauses, jnp.log(sm), 0.0)
        o_ref[0, 0] = -jnp.sum(term)

    out = pl.pallas_call(
        body,
        out_shape=jax.ShapeDtypeStruct((1, 1), jnp.float32),
        out_specs=pl.BlockSpec(memory_space=pltpu.SMEM),
    )(num2, den2)
    return out[0, 0]


def kernel(xv, adj_pos, adj_neg):
    x = xv.reshape(-1)
    v_nodes = x.shape[0]
    num_clauses = v_nodes
    e_edges = adj_pos.shape[1]
    assert adj_neg.shape[1] == e_edges

    n_pad = ((num_clauses + _NS * 16 - 1) // (_NS * 16)) * (_NS * 16)
    if n_pad == num_clauses:
        n_pad += _NS * 16
    ep = ((e_edges + _NS * _W - 1) // (_NS * _W)) * (_NS * _W)
    windows_per_tile = ep // (_NS * _W)
    pads = ep - e_edges

    spare = n_pad - num_clauses
    pad_c = num_clauses + (jnp.arange(pads, dtype=jnp.int32) % spare)
    pad_v = jnp.arange(pads, dtype=jnp.int32) % v_nodes
    rows = ep // _LANE
    c2 = jnp.stack([
        jnp.concatenate([adj_pos[0], pad_c]),
        jnp.concatenate([adj_neg[0], pad_c]),
    ]).reshape(2, rows, _LANE)
    v2 = jnp.stack([
        jnp.concatenate([adj_pos[1], pad_v]),
        jnp.concatenate([adj_neg[1], pad_v]),
    ]).reshape(2, rows, _LANE)

    num_flat, den_flat = _sc_segment_sums(x, c2, v2, windows_per_tile, n_pad)
    return _tc_loss(num_flat.reshape(_NC, n_pad), den_flat.reshape(_NC, n_pad),
                    num_clauses)


# speedup vs baseline: 126.3927x; 1.3774x over previous
import dataclasses
import functools

import jax
import jax.numpy as jnp
from jax import lax
from jax.experimental import pallas as pl
from jax.experimental.pallas import tpu as pltpu
from jax.experimental.pallas import tpu_sc as plsc

_P = 5.0
_A = 10.0

_NC = 2
_NS = 16
_LANE = 128
_WROWS = 16
_W = _WROWS * _LANE


def _sc_segment_sums(x, c2, v2, windows_per_tile, n_pad):
    v_nodes = x.shape[0]
    bins_per_tile = n_pad // _NS
    mesh = plsc.VectorSubcoreMesh(core_axis_name="c", subcore_axis_name="s")
    cp = pltpu.CompilerParams()
    if "needs_layout_passes" in pltpu.CompilerParams.__dataclass_fields__:
        cp = dataclasses.replace(cp, needs_layout_passes=False)

    @functools.partial(
        pl.kernel,
        out_type=(
            jax.ShapeDtypeStruct((_NC * n_pad,), jnp.float32),
            jax.ShapeDtypeStruct((_NC * n_pad,), jnp.float32),
        ),
        mesh=mesh,
        compiler_params=cp,
        scratch_types=[
            pltpu.VMEM((v_nodes,), jnp.float32),
            pltpu.VMEM((_WROWS, _LANE), jnp.int32),
            pltpu.VMEM((_WROWS, _LANE), jnp.int32),
            pltpu.VMEM((_WROWS, _LANE), jnp.float32),
            pltpu.VMEM((_WROWS, _LANE), jnp.float32),
            pltpu.VMEM((bins_per_tile,), jnp.float32),
            pltpu.VMEM_SHARED((n_pad,), jnp.float32),
            pltpu.VMEM_SHARED((n_pad,), jnp.float32),
            pltpu.SemaphoreType.DMA,
        ],
    )
    def k(x_hbm, c_hbm, v_hbm, num_out, den_out,
          x_v, cidx, vidx, nbuf, ebuf, zbuf, num_sh, den_sh, sem):
        c = lax.axis_index("c")
        s = lax.axis_index("s")

        @pl.loop(0, bins_per_tile, step=16)
        def _(i):
            zbuf[pl.ds(i, 16)] = jnp.zeros((16,), jnp.float32)

        pltpu.sync_copy(zbuf, num_sh.at[pl.ds(s * bins_per_tile, bins_per_tile)])
        pltpu.sync_copy(zbuf, den_sh.at[pl.ds(s * bins_per_tile, bins_per_tile)])

        pltpu.sync_copy(x_hbm, x_v)
        plsc.subcore_barrier()

        def process(is_neg):
            phase = jnp.where(is_neg, 1, 0)

            @pl.loop(0, windows_per_tile)
            def _(w):
                row0 = (s * windows_per_tile + w) * _WROWS
                pltpu.sync_copy(c_hbm.at[phase, pl.ds(row0, _WROWS)], cidx)
                pltpu.sync_copy(v_hbm.at[phase, pl.ds(row0, _WROWS)], vidx)

                @pl.loop(0, _WROWS)
                def _(r):
                    @pl.loop(0, _LANE, step=16)
                    def _(i):
                        vi = vidx[r, pl.ds(i, 16)]
                        xg = plsc.load_gather(x_v, [vi])
                        lit = jnp.where(is_neg, 1.0 - xg, xg)
                        e = jnp.exp(lit * _P)
                        nbuf[r, pl.ds(i, 16)] = lit * e
                        ebuf[r, pl.ds(i, 16)] = e

                    pltpu.async_copy(nbuf.at[r], num_sh.at[cidx.at[r]], sem,
                                     add=True)
                    pltpu.async_copy(ebuf.at[r], den_sh.at[cidx.at[r]], sem,
                                     add=True)

                @pl.loop(0, _WROWS)
                def _(r):
                    pltpu.make_async_copy(nbuf.at[r], num_sh.at[cidx.at[r]],
                                          sem).wait()
                    pltpu.make_async_copy(ebuf.at[r], den_sh.at[cidx.at[r]],
                                          sem).wait()

        process(c == 1)

        plsc.subcore_barrier()
        base = c * n_pad + s * bins_per_tile
        pltpu.sync_copy(num_sh.at[pl.ds(s * bins_per_tile, bins_per_tile)], zbuf)
        pltpu.sync_copy(zbuf, num_out.at[pl.ds(base, bins_per_tile)])
        pltpu.sync_copy(den_sh.at[pl.ds(s * bins_per_tile, bins_per_tile)], zbuf)
        pltpu.sync_copy(zbuf, den_out.at[pl.ds(base, bins_per_tile)])

    return k(x, c2, v2)


def _tc_loss(num2, den2, num_clauses):
    def body(n_ref, d_ref, o_ref):
        n = n_ref[0:1, :] + n_ref[1:2, :]
        d = d_ref[0:1, :] + d_ref[1:2, :]
        r = n / d
        sm = 1.0 / (1.0 + jnp.exp(_A * (0.5 - r)))
        col = lax.broadcasted_iota(jnp.int32, sm.shape, 1)
        term = jnp.where(col < num_clauses, jnp.log(sm), 0.0)
        o_ref[0, 0] = -jnp.sum(term)

    out = pl.pallas_call(
        body,
        out_shape=jax.ShapeDtypeStruct((1, 1), jnp.float32),
        out_specs=pl.BlockSpec(memory_space=pltpu.SMEM),
    )(num2, den2)
    return out[0, 0]


def kernel(xv, adj_pos, adj_neg):
    x = xv.reshape(-1)
    v_nodes = x.shape[0]
    num_clauses = v_nodes
    e_edges = adj_pos.shape[1]
    assert adj_neg.shape[1] == e_edges

    n_pad = ((num_clauses + _NS * 16 - 1) // (_NS * 16)) * (_NS * 16)
    if n_pad == num_clauses:
        n_pad += _NS * 16
    ep = ((e_edges + _NS * _W - 1) // (_NS * _W)) * (_NS * _W)
    windows_per_tile = ep // (_NS * _W)
    pads = ep - e_edges

    spare = n_pad - num_clauses
    pad_c = num_clauses + (jnp.arange(pads, dtype=jnp.int32) % spare)
    pad_v = jnp.arange(pads, dtype=jnp.int32) % v_nodes
    rows = ep // _LANE
    c2 = jnp.stack([
        jnp.concatenate([adj_pos[0], pad_c]),
        jnp.concatenate([adj_neg[0], pad_c]),
    ]).reshape(2, rows, _LANE)
    v2 = jnp.stack([
        jnp.concatenate([adj_pos[1], pad_v]),
        jnp.concatenate([adj_neg[1], pad_v]),
    ]).reshape(2, rows, _LANE)

    num_flat, den_flat = _sc_segment_sums(x, c2, v2, windows_per_tile, n_pad)
    return _tc_loss(num_flat.reshape(_NC, n_pad), den_flat.reshape(_NC, n_pad),
                    num_clauses)


# speedup vs baseline: 138.8918x; 1.0989x over previous
import dataclasses
import functools

import jax
import jax.numpy as jnp
from jax import lax
from jax.experimental import pallas as pl
from jax.experimental.pallas import tpu as pltpu
from jax.experimental.pallas import tpu_sc as plsc

_P = 5.0
_A = 10.0

_NC = 2
_NS = 16
_LANE = 128
_WROWS = 16
_W = _WROWS * _LANE


def _sc_segment_sums(x, c2, v2, windows_per_tile, n_pad):
    v_nodes = x.shape[0]
    bins_per_tile = n_pad // _NS
    mesh = plsc.VectorSubcoreMesh(core_axis_name="c", subcore_axis_name="s")
    cp = pltpu.CompilerParams()
    if "needs_layout_passes" in pltpu.CompilerParams.__dataclass_fields__:
        cp = dataclasses.replace(cp, needs_layout_passes=False)

    @functools.partial(
        pl.kernel,
        out_type=(
            jax.ShapeDtypeStruct((_NC * n_pad,), jnp.float32),
            jax.ShapeDtypeStruct((_NC * n_pad,), jnp.float32),
        ),
        mesh=mesh,
        compiler_params=cp,
        scratch_types=[
            pltpu.VMEM((v_nodes,), jnp.float32),
            pltpu.VMEM((2, _WROWS, _LANE), jnp.int32),
            pltpu.VMEM((2, _WROWS, _LANE), jnp.int32),
            pltpu.VMEM((2, _WROWS, _LANE), jnp.float32),
            pltpu.VMEM((2, _WROWS, _LANE), jnp.float32),
            pltpu.VMEM((bins_per_tile,), jnp.float32),
            pltpu.VMEM_SHARED((n_pad,), jnp.float32),
            pltpu.VMEM_SHARED((n_pad,), jnp.float32),
            pltpu.SemaphoreType.DMA,
            pltpu.SemaphoreType.DMA,
        ],
    )
    def k(x_hbm, c_hbm, v_hbm, num_out, den_out,
          x_v, cidx, vidx, nbuf, ebuf, zbuf, num_sh, den_sh, sem, sem_in):
        c = lax.axis_index("c")
        s = lax.axis_index("s")

        @pl.loop(0, bins_per_tile, step=16)
        def _(i):
            zbuf[pl.ds(i, 16)] = jnp.zeros((16,), jnp.float32)

        pltpu.sync_copy(zbuf, num_sh.at[pl.ds(s * bins_per_tile, bins_per_tile)])
        pltpu.sync_copy(zbuf, den_sh.at[pl.ds(s * bins_per_tile, bins_per_tile)])

        pltpu.sync_copy(x_hbm, x_v)
        plsc.subcore_barrier()

        def process(is_neg):
            phase = jnp.where(is_neg, 1, 0)

            def fire_in(w, slot):
                row0 = (s * windows_per_tile + w) * _WROWS
                pltpu.async_copy(c_hbm.at[phase, pl.ds(row0, _WROWS)],
                                 cidx.at[slot], sem_in)
                pltpu.async_copy(v_hbm.at[phase, pl.ds(row0, _WROWS)],
                                 vidx.at[slot], sem_in)

            def wait_in(w, slot):
                row0 = (s * windows_per_tile + w) * _WROWS
                pltpu.make_async_copy(c_hbm.at[phase, pl.ds(row0, _WROWS)],
                                      cidx.at[slot], sem_in).wait()
                pltpu.make_async_copy(v_hbm.at[phase, pl.ds(row0, _WROWS)],
                                      vidx.at[slot], sem_in).wait()

            def drain_scatters(slot):
                @pl.loop(0, _WROWS)
                def _(r):
                    pltpu.make_async_copy(nbuf.at[slot, r],
                                          num_sh.at[cidx.at[slot, r]],
                                          sem).wait()
                    pltpu.make_async_copy(ebuf.at[slot, r],
                                          den_sh.at[cidx.at[slot, r]],
                                          sem).wait()

            fire_in(0, 0)

            @pl.loop(0, windows_per_tile)
            def _(w):
                slot = lax.rem(w, 2)

                @pl.when(w >= 2)
                def _():
                    drain_scatters(slot)

                wait_in(w, slot)

                @pl.when(w < windows_per_tile - 1)
                def _():
                    fire_in(w + 1, 1 - slot)

                @pl.loop(0, _WROWS)
                def _(r):
                    @pl.loop(0, _LANE, step=16)
                    def _(i):
                        vi = vidx[slot, r, pl.ds(i, 16)]
                        xg = plsc.load_gather(x_v, [vi])
                        lit = jnp.where(is_neg, 1.0 - xg, xg)
                        e = jnp.exp(lit * _P)
                        nbuf[slot, r, pl.ds(i, 16)] = lit * e
                        ebuf[slot, r, pl.ds(i, 16)] = e

                    pltpu.async_copy(nbuf.at[slot, r],
                                     num_sh.at[cidx.at[slot, r]], sem,
                                     add=True)
                    pltpu.async_copy(ebuf.at[slot, r],
                                     den_sh.at[cidx.at[slot, r]], sem,
                                     add=True)

            if windows_per_tile >= 2:
                drain_scatters(lax.rem(windows_per_tile - 2, 2))
            drain_scatters(lax.rem(windows_per_tile - 1, 2))

        process(c == 1)

        plsc.subcore_barrier()
        base = c * n_pad + s * bins_per_tile
        pltpu.sync_copy(num_sh.at[pl.ds(s * bins_per_tile, bins_per_tile)], zbuf)
        pltpu.sync_copy(zbuf, num_out.at[pl.ds(base, bins_per_tile)])
        pltpu.sync_copy(den_sh.at[pl.ds(s * bins_per_tile, bins_per_tile)], zbuf)
        pltpu.sync_copy(zbuf, den_out.at[pl.ds(base, bins_per_tile)])

    return k(x, c2, v2)


def _tc_loss(num2, den2, num_clauses):
    def body(n_ref, d_ref, o_ref):
        n = n_ref[0:1, :] + n_ref[1:2, :]
        d = d_ref[0:1, :] + d_ref[1:2, :]
        r = n / d
        sm = 1.0 / (1.0 + jnp.exp(_A * (0.5 - r)))
        col = lax.broadcasted_iota(jnp.int32, sm.shape, 1)
        term = jnp.where(col < num_clauses, jnp.log(sm), 0.0)
        o_ref[0, 0] = -jnp.sum(term)

    out = pl.pallas_call(
        body,
        out_shape=jax.ShapeDtypeStruct((1, 1), jnp.float32),
        out_specs=pl.BlockSpec(memory_space=pltpu.SMEM),
    )(num2, den2)
    return out[0, 0]


def kernel(xv, adj_pos, adj_neg):
    x = xv.reshape(-1)
    v_nodes = x.shape[0]
    num_clauses = v_nodes
    e_edges = adj_pos.shape[1]
    assert adj_neg.shape[1] == e_edges

    n_pad = ((num_clauses + _NS * 16 - 1) // (_NS * 16)) * (_NS * 16)
    if n_pad == num_clauses:
        n_pad += _NS * 16
    ep = ((e_edges + _NS * _W - 1) // (_NS * _W)) * (_NS * _W)
    windows_per_tile = ep // (_NS * _W)
    pads = ep - e_edges

    spare = n_pad - num_clauses
    pad_c = num_clauses + (jnp.arange(pads, dtype=jnp.int32) % spare)
    pad_v = jnp.arange(pads, dtype=jnp.int32) % v_nodes
    rows = ep // _LANE
    c2 = jnp.stack([
        jnp.concatenate([adj_pos[0], pad_c]),
        jnp.concatenate([adj_neg[0], pad_c]),
    ]).reshape(2, rows, _LANE)
    v2 = jnp.stack([
        jnp.concatenate([adj_pos[1], pad_v]),
        jnp.concatenate([adj_neg[1], pad_v]),
    ]).reshape(2, rows, _LANE)

    num_flat, den_flat = _sc_segment_sums(x, c2, v2, windows_per_tile, n_pad)
    return _tc_loss(num_flat.reshape(_NC, n_pad), den_flat.reshape(_NC, n_pad),
                    num_clauses)


# speedup vs baseline: 203.8591x; 1.4678x over previous
import dataclasses
import functools

import jax
import jax.numpy as jnp
from jax import lax
from jax.experimental import pallas as pl
from jax.experimental.pallas import tpu as pltpu
from jax.experimental.pallas import tpu_sc as plsc

_P = 5.0
_A = 10.0

_NC = 2
_NS = 16
_LANE = 128
_WROWS = 16


def _sc_segment_sums(x, pos3, neg3, n_pad):
    v_nodes = x.shape[0]
    total_rows = pos3.shape[1]
    assert neg3.shape[1] == total_rows
    total_full = total_rows // _WROWS
    tail_rows = total_rows % _WROWS
    bins_per_tile = n_pad // _NS

    mesh = plsc.VectorSubcoreMesh(core_axis_name="c", subcore_axis_name="s")
    cp = pltpu.CompilerParams()
    if "needs_layout_passes" in pltpu.CompilerParams.__dataclass_fields__:
        cp = dataclasses.replace(cp, needs_layout_passes=False)

    @functools.partial(
        pl.kernel,
        out_type=(
            jax.ShapeDtypeStruct((_NC * n_pad,), jnp.float32),
            jax.ShapeDtypeStruct((_NC * n_pad,), jnp.float32),
        ),
        mesh=mesh,
        compiler_params=cp,
        scratch_types=[
            pltpu.VMEM((v_nodes,), jnp.float32),
            pltpu.VMEM((2, _WROWS, _LANE), jnp.int32),
            pltpu.VMEM((2, _WROWS, _LANE), jnp.int32),
            pltpu.VMEM((2, _WROWS, _LANE), jnp.float32),
            pltpu.VMEM((2, _WROWS, _LANE), jnp.float32),
            pltpu.VMEM((bins_per_tile,), jnp.float32),
            pltpu.VMEM_SHARED((n_pad,), jnp.float32),
            pltpu.VMEM_SHARED((n_pad,), jnp.float32),
            pltpu.SemaphoreType.DMA,
            pltpu.SemaphoreType.DMA,
        ],
    )
    def k(x_hbm, pos_hbm, neg_hbm, num_out, den_out,
          x_v, cidx, vidx, nbuf, ebuf, zbuf, num_sh, den_sh, sem, sem_in):
        c = lax.axis_index("c")
        s = lax.axis_index("s")

        @pl.loop(0, bins_per_tile, step=16)
        def _(i):
            zbuf[pl.ds(i, 16)] = jnp.zeros((16,), jnp.float32)

        pltpu.sync_copy(zbuf, num_sh.at[pl.ds(s * bins_per_tile, bins_per_tile)])
        pltpu.sync_copy(zbuf, den_sh.at[pl.ds(s * bins_per_tile, bins_per_tile)])

        pltpu.sync_copy(x_hbm, x_v)
        plsc.subcore_barrier()

        nchunks = lax.div(total_full - 1 - s, _NS) + 1

        def process(adj_hbm, is_neg):
            def row_base(kk):
                return (s + kk * _NS) * _WROWS

            def fire_in(kk, slot):
                rb = row_base(kk)
                pltpu.async_copy(adj_hbm.at[0, pl.ds(rb, _WROWS)],
                                 cidx.at[slot], sem_in)
                pltpu.async_copy(adj_hbm.at[1, pl.ds(rb, _WROWS)],
                                 vidx.at[slot], sem_in)

            def wait_in(kk, slot):
                rb = row_base(kk)
                pltpu.make_async_copy(adj_hbm.at[0, pl.ds(rb, _WROWS)],
                                      cidx.at[slot], sem_in).wait()
                pltpu.make_async_copy(adj_hbm.at[1, pl.ds(rb, _WROWS)],
                                      vidx.at[slot], sem_in).wait()

            def drain_scatters(slot):
                @pl.loop(0, _WROWS)
                def _(r):
                    pltpu.make_async_copy(nbuf.at[slot, r],
                                          num_sh.at[cidx.at[slot, r]],
                                          sem).wait()
                    pltpu.make_async_copy(ebuf.at[slot, r],
                                          den_sh.at[cidx.at[slot, r]],
                                          sem).wait()

            fire_in(0, 0)

            @pl.loop(0, nchunks)
            def _(kk):
                slot = lax.rem(kk, 2)

                @pl.when(kk >= 2)
                def _():
                    drain_scatters(slot)

                wait_in(kk, slot)

                @pl.when(kk < nchunks - 1)
                def _():
                    fire_in(kk + 1, 1 - slot)

                @pl.loop(0, _WROWS)
                def _(r):
                    @pl.loop(0, _LANE, step=16)
                    def _(i):
                        vi = vidx[slot, r, pl.ds(i, 16)]
                        xg = plsc.load_gather(x_v, [vi])
                        lit = (1.0 - xg) if is_neg else xg
                        e = jnp.exp(lit * _P)
                        nbuf[slot, r, pl.ds(i, 16)] = lit * e
                        ebuf[slot, r, pl.ds(i, 16)] = e

                    pltpu.async_copy(nbuf.at[slot, r],
                                     num_sh.at[cidx.at[slot, r]], sem,
                                     add=True)
                    pltpu.async_copy(ebuf.at[slot, r],
                                     den_sh.at[cidx.at[slot, r]], sem,
                                     add=True)

            @pl.when(nchunks >= 2)
            def _():
                drain_scatters(lax.rem(nchunks - 2, 2))

            drain_scatters(lax.rem(nchunks - 1, 2))

            if tail_rows:
                @pl.when(s == total_full % _NS)
                def _():
                    rb = total_full * _WROWS
                    pltpu.sync_copy(adj_hbm.at[0, pl.ds(rb, tail_rows)],
                                    cidx.at[0, pl.ds(0, tail_rows)])
                    pltpu.sync_copy(adj_hbm.at[1, pl.ds(rb, tail_rows)],
                                    vidx.at[0, pl.ds(0, tail_rows)])

                    @pl.loop(0, tail_rows)
                    def _(r):
                        @pl.loop(0, _LANE, step=16)
                        def _(i):
                            vi = vidx[0, r, pl.ds(i, 16)]
                            xg = plsc.load_gather(x_v, [vi])
                            lit = (1.0 - xg) if is_neg else xg
                            e = jnp.exp(lit * _P)
                            nbuf[0, r, pl.ds(i, 16)] = lit * e
                            ebuf[0, r, pl.ds(i, 16)] = e

                        pltpu.async_copy(nbuf.at[0, r],
                                         num_sh.at[cidx.at[0, r]], sem,
                                         add=True)
                        pltpu.async_copy(ebuf.at[0, r],
                                         den_sh.at[cidx.at[0, r]], sem,
                                         add=True)

                    @pl.loop(0, tail_rows)
                    def _(r):
                        pltpu.make_async_copy(nbuf.at[0, r],
                                              num_sh.at[cidx.at[0, r]],
                                              sem).wait()
                        pltpu.make_async_copy(ebuf.at[0, r],
                                              den_sh.at[cidx.at[0, r]],
                                              sem).wait()

        @pl.when(c == 0)
        def _():
            process(pos_hbm, False)

        @pl.when(c == 1)
        def _():
            process(neg_hbm, True)

        plsc.subcore_barrier()
        base = c * n_pad + s * bins_per_tile
        pltpu.sync_copy(num_sh.at[pl.ds(s * bins_per_tile, bins_per_tile)], zbuf)
        pltpu.sync_copy(zbuf, num_out.at[pl.ds(base, bins_per_tile)])
        pltpu.sync_copy(den_sh.at[pl.ds(s * bins_per_tile, bins_per_tile)], zbuf)
        pltpu.sync_copy(zbuf, den_out.at[pl.ds(base, bins_per_tile)])

    return k(x, pos3, neg3)


def _tc_loss(num2, den2, num_clauses):
    def body(n_ref, d_ref, o_ref):
        n = n_ref[0:1, :] + n_ref[1:2, :]
        d = d_ref[0:1, :] + d_ref[1:2, :]
        r = n / d
        sm = 1.0 / (1.0 + jnp.exp(_A * (0.5 - r)))
        col = lax.broadcasted_iota(jnp.int32, sm.shape, 1)
        term = jnp.where(col < num_clauses, jnp.log(sm), 0.0)
        o_ref[0, 0] = -jnp.sum(term)

    out = pl.pallas_call(
        body,
        out_shape=jax.ShapeDtypeStruct((1, 1), jnp.float32),
        out_specs=pl.BlockSpec(memory_space=pltpu.SMEM),
    )(num2, den2)
    return out[0, 0]


def kernel(xv, adj_pos, adj_neg):
    x = xv.reshape(-1)
    v_nodes = x.shape[0]
    num_clauses = v_nodes
    e_edges = adj_pos.shape[1]
    assert adj_neg.shape[1] == e_edges
    assert e_edges % _LANE == 0

    n_pad = ((num_clauses + _NS * 16 - 1) // (_NS * 16)) * (_NS * 16)
    if n_pad == num_clauses:
        n_pad += _NS * 16

    rows = e_edges // _LANE
    pos3 = adj_pos.reshape(2, rows, _LANE)
    neg3 = adj_neg.reshape(2, rows, _LANE)

    num_flat, den_flat = _sc_segment_sums(x, pos3, neg3, n_pad)
    return _tc_loss(num_flat.reshape(_NC, n_pad), den_flat.reshape(_NC, n_pad),
                    num_clauses)


# speedup vs baseline: 225.7546x; 1.1074x over previous
import dataclasses
import functools

import jax
import jax.numpy as jnp
from jax import lax
from jax.experimental import pallas as pl
from jax.experimental.pallas import tpu as pltpu
from jax.experimental.pallas import tpu_sc as plsc

_P = 5.0
_A = 10.0

_NC = 2
_NS = 16
_LANE = 128
_WROWS = 16


def _sc_segment_sums(x, pos, neg, n_pad):
    v_nodes = x.shape[0]
    total_rows = pos.shape[1] // _LANE
    total_full = total_rows // _WROWS
    tail_rows = total_rows % _WROWS
    bins_per_tile = n_pad // _NS

    mesh = plsc.VectorSubcoreMesh(core_axis_name="c", subcore_axis_name="s")
    cp = pltpu.CompilerParams()
    if "needs_layout_passes" in pltpu.CompilerParams.__dataclass_fields__:
        cp = dataclasses.replace(cp, needs_layout_passes=False)

    @functools.partial(
        pl.kernel,
        out_type=(
            jax.ShapeDtypeStruct((_NC * n_pad,), jnp.float32),
            jax.ShapeDtypeStruct((_NC * n_pad,), jnp.float32),
        ),
        mesh=mesh,
        compiler_params=cp,
        scratch_types=[
            pltpu.VMEM((v_nodes,), jnp.float32),
            pltpu.VMEM((2, _WROWS, _LANE), jnp.int32),
            pltpu.VMEM((2, _WROWS, _LANE), jnp.int32),
            pltpu.VMEM((2, _WROWS, _LANE), jnp.float32),
            pltpu.VMEM((2, _WROWS, _LANE), jnp.float32),
            pltpu.VMEM((bins_per_tile,), jnp.float32),
            pltpu.VMEM_SHARED((n_pad,), jnp.float32),
            pltpu.VMEM_SHARED((n_pad,), jnp.float32),
            pltpu.SemaphoreType.DMA,
            pltpu.SemaphoreType.DMA,
        ],
    )
    def k(x_hbm, pos_hbm, neg_hbm, num_out, den_out,
          x_v, cidx, vidx, nbuf, ebuf, zbuf, num_sh, den_sh, sem, sem_in):
        c = lax.axis_index("c")
        s = lax.axis_index("s")

        @pl.loop(0, bins_per_tile, step=16)
        def _(i):
            zbuf[pl.ds(i, 16)] = jnp.zeros((16,), jnp.float32)

        pltpu.sync_copy(zbuf, num_sh.at[pl.ds(s * bins_per_tile, bins_per_tile)])
        pltpu.sync_copy(zbuf, den_sh.at[pl.ds(s * bins_per_tile, bins_per_tile)])

        pltpu.sync_copy(x_hbm, x_v)
        plsc.subcore_barrier()

        nchunks = lax.div(total_full - 1 - s, _NS) + 1

        def process(adj_hbm, is_neg):
            def row_base(kk):
                return (s + kk * _NS) * _WROWS

            def fire_in(kk, slot):
                rb = row_base(kk)
                for r in range(_WROWS):
                    pltpu.async_copy(
                        adj_hbm.at[0, pl.ds((rb + r) * _LANE, _LANE)],
                        cidx.at[slot, r], sem_in)
                    pltpu.async_copy(
                        adj_hbm.at[1, pl.ds((rb + r) * _LANE, _LANE)],
                        vidx.at[slot, r], sem_in)

            def wait_in(kk, slot):
                rb = row_base(kk)
                for r in range(_WROWS):
                    pltpu.make_async_copy(
                        adj_hbm.at[0, pl.ds((rb + r) * _LANE, _LANE)],
                        cidx.at[slot, r], sem_in).wait()
                    pltpu.make_async_copy(
                        adj_hbm.at[1, pl.ds((rb + r) * _LANE, _LANE)],
                        vidx.at[slot, r], sem_in).wait()

            def drain_scatters(slot):
                @pl.loop(0, _WROWS)
                def _(r):
                    pltpu.make_async_copy(nbuf.at[slot, r],
                                          num_sh.at[cidx.at[slot, r]],
                                          sem).wait()
                    pltpu.make_async_copy(ebuf.at[slot, r],
                                          den_sh.at[cidx.at[slot, r]],
                                          sem).wait()

            fire_in(0, 0)

            @pl.loop(0, nchunks)
            def _(kk):
                slot = lax.rem(kk, 2)

                @pl.when(kk >= 2)
                def _():
                    drain_scatters(slot)

                wait_in(kk, slot)

                @pl.when(kk < nchunks - 1)
                def _():
                    fire_in(kk + 1, 1 - slot)

                @pl.loop(0, _WROWS)
                def _(r):
                    @pl.loop(0, _LANE, step=16)
                    def _(i):
                        vi = vidx[slot, r, pl.ds(i, 16)]
                        xg = plsc.load_gather(x_v, [vi])
                        lit = (1.0 - xg) if is_neg else xg
                        e = jnp.exp(lit * _P)
                        nbuf[slot, r, pl.ds(i, 16)] = lit * e
                        ebuf[slot, r, pl.ds(i, 16)] = e

                    pltpu.async_copy(nbuf.at[slot, r],
                                     num_sh.at[cidx.at[slot, r]], sem,
                                     add=True)
                    pltpu.async_copy(ebuf.at[slot, r],
                                     den_sh.at[cidx.at[slot, r]], sem,
                                     add=True)

            @pl.when(nchunks >= 2)
            def _():
                drain_scatters(lax.rem(nchunks - 2, 2))

            drain_scatters(lax.rem(nchunks - 1, 2))

            if tail_rows:
                @pl.when(s == total_full % _NS)
                def _():
                    rb = total_full * _WROWS
                    for r in range(tail_rows):
                        pltpu.sync_copy(
                            adj_hbm.at[0, pl.ds((rb + r) * _LANE, _LANE)],
                            cidx.at[0, r])
                        pltpu.sync_copy(
                            adj_hbm.at[1, pl.ds((rb + r) * _LANE, _LANE)],
                            vidx.at[0, r])

                    @pl.loop(0, tail_rows)
                    def _(r):
                        @pl.loop(0, _LANE, step=16)
                        def _(i):
                            vi = vidx[0, r, pl.ds(i, 16)]
                            xg = plsc.load_gather(x_v, [vi])
                            lit = (1.0 - xg) if is_neg else xg
                            e = jnp.exp(lit * _P)
                            nbuf[0, r, pl.ds(i, 16)] = lit * e
                            ebuf[0, r, pl.ds(i, 16)] = e

                        pltpu.async_copy(nbuf.at[0, r],
                                         num_sh.at[cidx.at[0, r]], sem,
                                         add=True)
                        pltpu.async_copy(ebuf.at[0, r],
                                         den_sh.at[cidx.at[0, r]], sem,
                                         add=True)

                    @pl.loop(0, tail_rows)
                    def _(r):
                        pltpu.make_async_copy(nbuf.at[0, r],
                                              num_sh.at[cidx.at[0, r]],
                                              sem).wait()
                        pltpu.make_async_copy(ebuf.at[0, r],
                                              den_sh.at[cidx.at[0, r]],
                                              sem).wait()

        @pl.when(c == 0)
        def _():
            process(pos_hbm, False)

        @pl.when(c == 1)
        def _():
            process(neg_hbm, True)

        plsc.subcore_barrier()
        base = c * n_pad + s * bins_per_tile
        pltpu.sync_copy(num_sh.at[pl.ds(s * bins_per_tile, bins_per_tile)], zbuf)
        pltpu.sync_copy(zbuf, num_out.at[pl.ds(base, bins_per_tile)])
        pltpu.sync_copy(den_sh.at[pl.ds(s * bins_per_tile, bins_per_tile)], zbuf)
        pltpu.sync_copy(zbuf, den_out.at[pl.ds(base, bins_per_tile)])

    return k(x, pos, neg)


def _tc_loss(num_flat, den_flat, n_pad, num_clauses):
    rows = n_pad // 128

    def body(n_ref, d_ref, o_ref):
        n = (n_ref[pl.ds(0, n_pad)] + n_ref[pl.ds(n_pad, n_pad)]).reshape(
            rows, 128)
        d = (d_ref[pl.ds(0, n_pad)] + d_ref[pl.ds(n_pad, n_pad)]).reshape(
            rows, 128)
        r = n / d
        sm = 1.0 / (1.0 + jnp.exp(_A * (0.5 - r)))
        idx = (lax.broadcasted_iota(jnp.int32, (rows, 128), 0) * 128
               + lax.broadcasted_iota(jnp.int32, (rows, 128), 1))
        term = jnp.where(idx < num_clauses, jnp.log(sm), 0.0)
        o_ref[0, 0] = -jnp.sum(term)

    out = pl.pallas_call(
        body,
        out_shape=jax.ShapeDtypeStruct((1, 1), jnp.float32),
        out_specs=pl.BlockSpec(memory_space=pltpu.SMEM),
    )(num_flat, den_flat)
    return out[0, 0]


def kernel(xv, adj_pos, adj_neg):
    x = xv.reshape(-1)
    v_nodes = x.shape[0]
    num_clauses = v_nodes
    e_edges = adj_pos.shape[1]
    assert adj_neg.shape[1] == e_edges
    assert e_edges % _LANE == 0

    n_pad = ((num_clauses + _NS * 16 - 1) // (_NS * 16)) * (_NS * 16)
    if n_pad == num_clauses:
        n_pad += _NS * 16

    num_flat, den_flat = _sc_segment_sums(x, adj_pos, adj_neg, n_pad)
    return _tc_loss(num_flat, den_flat, n_pad, num_clauses)


# speedup vs baseline: 290.7018x; 1.2877x over previous
import dataclasses
import functools

import jax
import jax.numpy as jnp
from jax import lax
from jax.experimental import pallas as pl
from jax.experimental.pallas import tpu as pltpu
from jax.experimental.pallas import tpu_sc as plsc

_P = 5.0
_A = 10.0

_NC = 2
_NS = 16
_W = 1280


def _sc_segment_sums(x, pos, neg, n_pad):
    v_nodes = x.shape[0]
    e_edges = pos.shape[1]
    assert e_edges % _W == 0
    total_chunks = e_edges // _W
    bins_per_tile = n_pad // _NS

    mesh = plsc.VectorSubcoreMesh(core_axis_name="c", subcore_axis_name="s")
    cp = pltpu.CompilerParams()
    if "needs_layout_passes" in pltpu.CompilerParams.__dataclass_fields__:
        cp = dataclasses.replace(cp, needs_layout_passes=False)

    @functools.partial(
        pl.kernel,
        out_type=(
            jax.ShapeDtypeStruct((_NC * n_pad,), jnp.float32),
            jax.ShapeDtypeStruct((_NC * n_pad,), jnp.float32),
        ),
        mesh=mesh,
        compiler_params=cp,
        scratch_types=[
            pltpu.VMEM((v_nodes,), jnp.float32),
        ] + [pltpu.VMEM((_W,), jnp.int32) for _ in range(6)]
          + [pltpu.VMEM((_W,), jnp.float32) for _ in range(6)]
          + [
            pltpu.VMEM((bins_per_tile,), jnp.float32),
            pltpu.VMEM_SHARED((n_pad,), jnp.float32),
            pltpu.VMEM_SHARED((n_pad,), jnp.float32),
            pltpu.SemaphoreType.DMA,
            pltpu.SemaphoreType.DMA,
        ],
    )
    def k(x_hbm, pos_hbm, neg_hbm, num_out, den_out, x_v,
          ci0, ci1, ci2, vi0, vi1, vi2, nb0, nb1, nb2, eb0, eb1, eb2,
          zbuf, num_sh, den_sh, sem, sem_in):
        bufs = ((ci0, vi0, nb0, eb0), (ci1, vi1, nb1, eb1),
                (ci2, vi2, nb2, eb2))
        c = lax.axis_index("c")
        s = lax.axis_index("s")

        @pl.loop(0, bins_per_tile, step=16)
        def _(i):
            zbuf[pl.ds(i, 16)] = jnp.zeros((16,), jnp.float32)

        pltpu.sync_copy(zbuf, num_sh.at[pl.ds(s * bins_per_tile, bins_per_tile)])
        pltpu.sync_copy(zbuf, den_sh.at[pl.ds(s * bins_per_tile, bins_per_tile)])

        pltpu.sync_copy(x_hbm, x_v)
        plsc.subcore_barrier()

        nch = lax.div(total_chunks - 1 - s, _NS) + 1

        def process(adj_hbm, is_neg):
            def base(kk):
                return (s + kk * _NS) * _W

            def fire_in(kk, m):
                b = base(kk)
                for q in range(0, _W, 128):
                    pltpu.async_copy(adj_hbm.at[0, pl.ds(b + q, 128)],
                                     bufs[m][0].at[pl.ds(q, 128)], sem_in)
                    pltpu.async_copy(adj_hbm.at[1, pl.ds(b + q, 128)],
                                     bufs[m][1].at[pl.ds(q, 128)], sem_in)

            def wait_in(kk, m):
                b = base(kk)
                for q in range(0, _W, 128):
                    pltpu.make_async_copy(
                        adj_hbm.at[0, pl.ds(b + q, 128)],
                        bufs[m][0].at[pl.ds(q, 128)], sem_in).wait()
                    pltpu.make_async_copy(
                        adj_hbm.at[1, pl.ds(b + q, 128)],
                        bufs[m][1].at[pl.ds(q, 128)], sem_in).wait()

            def drain_scatters(m):
                ci, _, nb, eb = bufs[m]
                pltpu.make_async_copy(nb, num_sh.at[ci], sem).wait()
                pltpu.make_async_copy(eb, den_sh.at[ci], sem).wait()

            def compute_and_fire(m, kk):
                ci, vi_b, nb, eb = bufs[m]

                @pl.loop(0, _W, step=16)
                def _(i):
                    vi = vi_b[pl.ds(i, 16)]
                    xg = plsc.load_gather(x_v, [vi])
                    lit = (1.0 - xg) if is_neg else xg
                    e = jnp.exp(lit * _P)
                    nb[pl.ds(i, 16)] = lit * e
                    eb[pl.ds(i, 16)] = e

                pltpu.async_copy(nb, num_sh.at[ci], sem, add=True)
                pltpu.async_copy(eb, den_sh.at[ci], sem, add=True)

            fire_in(0, 0)

            @pl.loop(0, nch + 2, step=3)
            def _(kk):
                for p in range(3):
                    m, m1 = p, (p + 1) % 3
                    j = kk + p

                    @pl.when((j >= 2) & (j - 2 < nch))
                    def _():
                        drain_scatters(m1)

                    @pl.when(j + 1 < nch)
                    def _():
                        fire_in(j + 1, m1)

                    @pl.when(j < nch)
                    def _():
                        wait_in(j, m)
                        compute_and_fire(m, j)

        @pl.when(c == 0)
        def _():
            process(pos_hbm, False)

        @pl.when(c == 1)
        def _():
            process(neg_hbm, True)

        plsc.subcore_barrier()
        base = c * n_pad + s * bins_per_tile
        pltpu.sync_copy(num_sh.at[pl.ds(s * bins_per_tile, bins_per_tile)], zbuf)
        pltpu.sync_copy(zbuf, num_out.at[pl.ds(base, bins_per_tile)])
        pltpu.sync_copy(den_sh.at[pl.ds(s * bins_per_tile, bins_per_tile)], zbuf)
        pltpu.sync_copy(zbuf, den_out.at[pl.ds(base, bins_per_tile)])

    return k(x, pos, neg)


def _tc_loss(num_flat, den_flat, n_pad, num_clauses):
    rows = n_pad // 128

    def body(n_ref, d_ref, o_ref):
        n = (n_ref[pl.ds(0, n_pad)] + n_ref[pl.ds(n_pad, n_pad)]).reshape(
            rows, 128)
        d = (d_ref[pl.ds(0, n_pad)] + d_ref[pl.ds(n_pad, n_pad)]).reshape(
            rows, 128)
        r = n / d
        sm = 1.0 / (1.0 + jnp.exp(_A * (0.5 - r)))
        idx = (lax.broadcasted_iota(jnp.int32, (rows, 128), 0) * 128
               + lax.broadcasted_iota(jnp.int32, (rows, 128), 1))
        term = jnp.where(idx < num_clauses, jnp.log(sm), 0.0)
        o_ref[0, 0] = -jnp.sum(term)

    out = pl.pallas_call(
        body,
        out_shape=jax.ShapeDtypeStruct((1, 1), jnp.float32),
        out_specs=pl.BlockSpec(memory_space=pltpu.SMEM),
    )(num_flat, den_flat)
    return out[0, 0]


def kernel(xv, adj_pos, adj_neg):
    x = xv.reshape(-1)
    v_nodes = x.shape[0]
    num_clauses = v_nodes
    e_edges = adj_pos.shape[1]
    assert adj_neg.shape[1] == e_edges
    assert e_edges % _W == 0

    n_pad = ((num_clauses + _NS * 16 - 1) // (_NS * 16)) * (_NS * 16)
    if n_pad == num_clauses:
        n_pad += _NS * 16

    num_flat, den_flat = _sc_segment_sums(x, adj_pos, adj_neg, n_pad)
    return _tc_loss(num_flat, den_flat, n_pad, num_clauses)


# speedup vs baseline: 399.0330x; 1.3727x over previous
import dataclasses
import functools

import jax
import jax.numpy as jnp
from jax import lax
from jax.experimental import pallas as pl
from jax.experimental.pallas import tpu as pltpu
from jax.experimental.pallas import tpu_sc as plsc

_P = 5.0
_A = 10.0

_NC = 2
_NS = 16
_W = 1280


def _sc_segment_sums(x, pos, neg, n_pad):
    v_nodes = x.shape[0]
    e_edges = pos.shape[1]
    assert e_edges % _W == 0
    total_chunks = e_edges // _W
    bins_per_tile = n_pad // _NS

    mesh = plsc.VectorSubcoreMesh(core_axis_name="c", subcore_axis_name="s")
    cp = pltpu.CompilerParams()
    if "needs_layout_passes" in pltpu.CompilerParams.__dataclass_fields__:
        cp = dataclasses.replace(cp, needs_layout_passes=False)

    @functools.partial(
        pl.kernel,
        out_type=(
            jax.ShapeDtypeStruct((_NC * n_pad,), jnp.float32),
            jax.ShapeDtypeStruct((_NC * n_pad,), jnp.float32),
        ),
        mesh=mesh,
        compiler_params=cp,
        scratch_types=[
            pltpu.VMEM((v_nodes,), jnp.float32),
        ] + [pltpu.VMEM((_W,), jnp.int32) for _ in range(6)]
          + [pltpu.VMEM((_W,), jnp.float32) for _ in range(6)]
          + [
            pltpu.VMEM((bins_per_tile,), jnp.float32),
            pltpu.VMEM_SHARED((n_pad,), jnp.float32),
            pltpu.VMEM_SHARED((n_pad,), jnp.float32),
            pltpu.SemaphoreType.DMA,
            pltpu.SemaphoreType.DMA,
        ],
    )
    def k(x_hbm, pos_hbm, neg_hbm, num_out, den_out, x_v,
          ci0, ci1, ci2, vi0, vi1, vi2, nb0, nb1, nb2, eb0, eb1, eb2,
          zbuf, num_sh, den_sh, sem, sem_in):
        bufs = ((ci0, vi0, nb0, eb0), (ci1, vi1, nb1, eb1),
                (ci2, vi2, nb2, eb2))
        c = lax.axis_index("c")
        s = lax.axis_index("s")

        @pl.loop(0, bins_per_tile, step=16)
        def _(i):
            zbuf[pl.ds(i, 16)] = jnp.zeros((16,), jnp.float32)

        pltpu.sync_copy(zbuf, num_sh.at[pl.ds(s * bins_per_tile, bins_per_tile)])
        pltpu.sync_copy(zbuf, den_sh.at[pl.ds(s * bins_per_tile, bins_per_tile)])

        pltpu.sync_copy(x_hbm, x_v)
        plsc.subcore_barrier()

        nch = lax.div(total_chunks - 1 - s, _NS) + 1

        def process(adj_hbm, is_neg):
            def base(kk):
                return (s + kk * _NS) * _W

            def fire_in(kk, m):
                b = base(kk)
                for q in range(0, _W, 128):
                    pltpu.async_copy(adj_hbm.at[0, pl.ds(b + q, 128)],
                                     bufs[m][0].at[pl.ds(q, 128)], sem_in)
                    pltpu.async_copy(adj_hbm.at[1, pl.ds(b + q, 128)],
                                     bufs[m][1].at[pl.ds(q, 128)], sem_in)

            def wait_in(kk, m):
                b = base(kk)
                for q in range(0, _W, 128):
                    pltpu.make_async_copy(
                        adj_hbm.at[0, pl.ds(b + q, 128)],
                        bufs[m][0].at[pl.ds(q, 128)], sem_in).wait()
                    pltpu.make_async_copy(
                        adj_hbm.at[1, pl.ds(b + q, 128)],
                        bufs[m][1].at[pl.ds(q, 128)], sem_in).wait()

            def drain_scatters(m):
                ci, _, nb, eb = bufs[m]
                pltpu.make_async_copy(nb, num_sh.at[ci], sem).wait()
                pltpu.make_async_copy(eb, den_sh.at[ci], sem).wait()

            def compute_and_fire(m, kk):
                ci, vi_b, nb, eb = bufs[m]

                @plsc.parallel_loop(0, _W, step=16, unroll=4)
                def _(i):
                    vi = vi_b[pl.ds(i, 16)]
                    xg = plsc.load_gather(x_v, [vi])
                    lit = (1.0 - xg) if is_neg else xg
                    e = jnp.exp(lit * _P)
                    nb[pl.ds(i, 16)] = lit * e
                    eb[pl.ds(i, 16)] = e

                pltpu.async_copy(nb, num_sh.at[ci], sem, add=True)
                pltpu.async_copy(eb, den_sh.at[ci], sem, add=True)

            fire_in(0, 0)

            @pl.loop(0, nch + 2, step=3)
            def _(kk):
                for p in range(3):
                    m, m1 = p, (p + 1) % 3
                    j = kk + p

                    @pl.when((j >= 2) & (j - 2 < nch))
                    def _():
                        drain_scatters(m1)

                    @pl.when(j + 1 < nch)
                    def _():
                        fire_in(j + 1, m1)

                    @pl.when(j < nch)
                    def _():
                        wait_in(j, m)
                        compute_and_fire(m, j)

        @pl.when(c == 0)
        def _():
            process(pos_hbm, False)

        @pl.when(c == 1)
        def _():
            process(neg_hbm, True)

        plsc.subcore_barrier()
        base = c * n_pad + s * bins_per_tile
        pltpu.sync_copy(num_sh.at[pl.ds(s * bins_per_tile, bins_per_tile)], zbuf)
        pltpu.sync_copy(zbuf, num_out.at[pl.ds(base, bins_per_tile)])
        pltpu.sync_copy(den_sh.at[pl.ds(s * bins_per_tile, bins_per_tile)], zbuf)
        pltpu.sync_copy(zbuf, den_out.at[pl.ds(base, bins_per_tile)])

    return k(x, pos, neg)


def _tc_loss(num_flat, den_flat, n_pad, num_clauses):
    rows = n_pad // 128

    def body(n_ref, d_ref, o_ref):
        n = (n_ref[pl.ds(0, n_pad)] + n_ref[pl.ds(n_pad, n_pad)]).reshape(
            rows, 128)
        d = (d_ref[pl.ds(0, n_pad)] + d_ref[pl.ds(n_pad, n_pad)]).reshape(
            rows, 128)
        r = n / d
        sm = 1.0 / (1.0 + jnp.exp(_A * (0.5 - r)))
        idx = (lax.broadcasted_iota(jnp.int32, (rows, 128), 0) * 128
               + lax.broadcasted_iota(jnp.int32, (rows, 128), 1))
        term = jnp.where(idx < num_clauses, jnp.log(sm), 0.0)
        o_ref[0, 0] = -jnp.sum(term)

    out = pl.pallas_call(
        body,
        out_shape=jax.ShapeDtypeStruct((1, 1), jnp.float32),
        out_specs=pl.BlockSpec(memory_space=pltpu.SMEM),
    )(num_flat, den_flat)
    return out[0, 0]


def kernel(xv, adj_pos, adj_neg):
    x = xv.reshape(-1)
    v_nodes = x.shape[0]
    num_clauses = v_nodes
    e_edges = adj_pos.shape[1]
    assert adj_neg.shape[1] == e_edges
    assert e_edges % _W == 0

    n_pad = ((num_clauses + _NS * 16 - 1) // (_NS * 16)) * (_NS * 16)
    if n_pad == num_clauses:
        n_pad += _NS * 16

    num_flat, den_flat = _sc_segment_sums(x, adj_pos, adj_neg, n_pad)
    return _tc_loss(num_flat, den_flat, n_pad, num_clauses)


# speedup vs baseline: 401.8274x; 1.0070x over previous
import dataclasses
import functools

import jax
import jax.numpy as jnp
from jax import lax
from jax.experimental import pallas as pl
from jax.experimental.pallas import tpu as pltpu
from jax.experimental.pallas import tpu_sc as plsc

_P = 5.0
_A = 10.0

_NC = 2
_NS = 16
_W = 1280


def _sc_segment_sums(x, pos, neg, n_pad):
    v_nodes = x.shape[0]
    e_edges = pos.shape[1]
    assert e_edges % _W == 0
    total_chunks = e_edges // _W
    bins_per_tile = n_pad // _NS

    mesh = plsc.VectorSubcoreMesh(core_axis_name="c", subcore_axis_name="s")
    cp = pltpu.CompilerParams()
    if "needs_layout_passes" in pltpu.CompilerParams.__dataclass_fields__:
        cp = dataclasses.replace(cp, needs_layout_passes=False)

    @functools.partial(
        pl.kernel,
        out_type=(
            jax.ShapeDtypeStruct((_NC * n_pad,), jnp.float32),
            jax.ShapeDtypeStruct((_NC * n_pad,), jnp.float32),
        ),
        mesh=mesh,
        compiler_params=cp,
        scratch_types=[
            pltpu.VMEM((v_nodes,), jnp.float32),
        ] + [pltpu.VMEM((_W,), jnp.int32) for _ in range(6)]
          + [pltpu.VMEM((_W,), jnp.float32) for _ in range(6)]
          + [
            pltpu.VMEM((bins_per_tile,), jnp.float32),
            pltpu.VMEM_SHARED((n_pad,), jnp.float32),
            pltpu.VMEM_SHARED((n_pad,), jnp.float32),
            pltpu.SemaphoreType.DMA,
            pltpu.SemaphoreType.DMA,
        ],
    )
    def k(x_hbm, pos_hbm, neg_hbm, num_out, den_out, x_v,
          ci0, ci1, ci2, vi0, vi1, vi2, nb0, nb1, nb2, eb0, eb1, eb2,
          zbuf, num_sh, den_sh, sem, sem_in):
        bufs = ((ci0, vi0, nb0, eb0), (ci1, vi1, nb1, eb1),
                (ci2, vi2, nb2, eb2))
        c = lax.axis_index("c")
        s = lax.axis_index("s")

        @plsc.parallel_loop(0, bins_per_tile, step=16, unroll=4)
        def _(i):
            zbuf[pl.ds(i, 16)] = jnp.zeros((16,), jnp.float32)

        pltpu.sync_copy(zbuf, num_sh.at[pl.ds(s * bins_per_tile, bins_per_tile)])
        pltpu.sync_copy(zbuf, den_sh.at[pl.ds(s * bins_per_tile, bins_per_tile)])

        pltpu.sync_copy(x_hbm, x_v)
        plsc.subcore_barrier()

        nch = lax.div(total_chunks - 1 - s, _NS) + 1

        def process(adj_hbm, is_neg):
            def base(kk):
                return (s + kk * _NS) * _W

            def fire_in(kk, m):
                b = base(kk)
                for q in range(0, _W, 128):
                    pltpu.async_copy(adj_hbm.at[0, pl.ds(b + q, 128)],
                                     bufs[m][0].at[pl.ds(q, 128)], sem_in)
                    pltpu.async_copy(adj_hbm.at[1, pl.ds(b + q, 128)],
                                     bufs[m][1].at[pl.ds(q, 128)], sem_in)

            def wait_in(kk, m):
                b = base(kk)
                for q in range(0, _W, 128):
                    pltpu.make_async_copy(
                        adj_hbm.at[0, pl.ds(b + q, 128)],
                        bufs[m][0].at[pl.ds(q, 128)], sem_in).wait()
                    pltpu.make_async_copy(
                        adj_hbm.at[1, pl.ds(b + q, 128)],
                        bufs[m][1].at[pl.ds(q, 128)], sem_in).wait()

            def drain_scatters(m):
                ci, _, nb, eb = bufs[m]
                pltpu.make_async_copy(nb, num_sh.at[ci], sem).wait()
                pltpu.make_async_copy(eb, den_sh.at[ci], sem).wait()

            def compute_and_fire(m, kk):
                ci, vi_b, nb, eb = bufs[m]

                @plsc.parallel_loop(0, _W, step=16, unroll=8)
                def _(i):
                    vi = vi_b[pl.ds(i, 16)]
                    xg = plsc.load_gather(x_v, [vi])
                    lit = (1.0 - xg) if is_neg else xg
                    e = jnp.exp(lit * _P)
                    nb[pl.ds(i, 16)] = lit * e
                    eb[pl.ds(i, 16)] = e

                pltpu.async_copy(nb, num_sh.at[ci], sem, add=True)
                pltpu.async_copy(eb, den_sh.at[ci], sem, add=True)

            fire_in(0, 0)

            @pl.loop(0, nch + 2, step=3)
            def _(kk):
                for p in range(3):
                    m, m1 = p, (p + 1) % 3
                    j = kk + p

                    @pl.when((j >= 2) & (j - 2 < nch))
                    def _():
                        drain_scatters(m1)

                    @pl.when(j + 1 < nch)
                    def _():
                        fire_in(j + 1, m1)

                    @pl.when(j < nch)
                    def _():
                        wait_in(j, m)
                        compute_and_fire(m, j)

        @pl.when(c == 0)
        def _():
            process(pos_hbm, False)

        @pl.when(c == 1)
        def _():
            process(neg_hbm, True)

        plsc.subcore_barrier()
        base = c * n_pad + s * bins_per_tile
        pltpu.sync_copy(num_sh.at[pl.ds(s * bins_per_tile, bins_per_tile)], zbuf)
        pltpu.sync_copy(zbuf, num_out.at[pl.ds(base, bins_per_tile)])
        pltpu.sync_copy(den_sh.at[pl.ds(s * bins_per_tile, bins_per_tile)], zbuf)
        pltpu.sync_copy(zbuf, den_out.at[pl.ds(base, bins_per_tile)])

    return k(x, pos, neg)


def _tc_loss(num_flat, den_flat, n_pad, num_clauses):
    rows = n_pad // 128

    def body(n_ref, d_ref, o_ref):
        n = (n_ref[pl.ds(0, n_pad)] + n_ref[pl.ds(n_pad, n_pad)]).reshape(
            rows, 128)
        d = (d_ref[pl.ds(0, n_pad)] + d_ref[pl.ds(n_pad, n_pad)]).reshape(
            rows, 128)
        r = n / d
        sm = 1.0 / (1.0 + jnp.exp(_A * (0.5 - r)))
        idx = (lax.broadcasted_iota(jnp.int32, (rows, 128), 0) * 128
               + lax.broadcasted_iota(jnp.int32, (rows, 128), 1))
        term = jnp.where(idx < num_clauses, jnp.log(sm), 0.0)
        o_ref[0, 0] = -jnp.sum(term)

    out = pl.pallas_call(
        body,
        out_shape=jax.ShapeDtypeStruct((1, 1), jnp.float32),
        out_specs=pl.BlockSpec(memory_space=pltpu.SMEM),
    )(num_flat, den_flat)
    return out[0, 0]


def kernel(xv, adj_pos, adj_neg):
    x = xv.reshape(-1)
    v_nodes = x.shape[0]
    num_clauses = v_nodes
    e_edges = adj_pos.shape[1]
    assert adj_neg.shape[1] == e_edges
    assert e_edges % _W == 0

    n_pad = ((num_clauses + _NS * 16 - 1) // (_NS * 16)) * (_NS * 16)
    if n_pad == num_clauses:
        n_pad += _NS * 16

    num_flat, den_flat = _sc_segment_sums(x, adj_pos, adj_neg, n_pad)
    return _tc_loss(num_flat, den_flat, n_pad, num_clauses)


# speedup vs baseline: 411.8919x; 1.0250x over previous
import dataclasses
import functools

import jax
import jax.numpy as jnp
from jax import lax
from jax.experimental import pallas as pl
from jax.experimental.pallas import tpu as pltpu
from jax.experimental.pallas import tpu_sc as plsc

_P = 5.0
_A = 10.0

_NC = 2
_NS = 16
_W = 1280


def _sc_segment_sums(x, pos, neg, n_pad):
    v_nodes = x.shape[0]
    e_edges = pos.shape[1]
    assert e_edges % _W == 0
    total_chunks = e_edges // _W
    bins_per_tile = n_pad // _NS

    mesh = plsc.VectorSubcoreMesh(core_axis_name="c", subcore_axis_name="s")
    cp = pltpu.CompilerParams()
    if "needs_layout_passes" in pltpu.CompilerParams.__dataclass_fields__:
        cp = dataclasses.replace(cp, needs_layout_passes=False)

    @functools.partial(
        pl.kernel,
        out_type=(
            jax.ShapeDtypeStruct((_NC * n_pad,), jnp.float32),
            jax.ShapeDtypeStruct((_NC * n_pad,), jnp.float32),
        ),
        mesh=mesh,
        compiler_params=cp,
        scratch_types=[
            pltpu.VMEM((v_nodes,), jnp.float32),
        ] + [pltpu.VMEM((_W,), jnp.int32) for _ in range(6)]
          + [pltpu.VMEM((_W,), jnp.float32) for _ in range(6)]
          + [
            pltpu.VMEM((bins_per_tile,), jnp.float32),
            pltpu.VMEM_SHARED((n_pad,), jnp.float32),
            pltpu.VMEM_SHARED((n_pad,), jnp.float32),
            pltpu.SemaphoreType.DMA,
            pltpu.SemaphoreType.DMA,
        ],
    )
    def k(x_hbm, pos_hbm, neg_hbm, num_out, den_out, x_v,
          ci0, ci1, ci2, vi0, vi1, vi2, nb0, nb1, nb2, eb0, eb1, eb2,
          zbuf, num_sh, den_sh, sem, sem_in):
        bufs = ((ci0, vi0, nb0, eb0), (ci1, vi1, nb1, eb1),
                (ci2, vi2, nb2, eb2))
        c = lax.axis_index("c")
        s = lax.axis_index("s")

        @plsc.parallel_loop(0, bins_per_tile, step=16, unroll=4)
        def _(i):
            zbuf[pl.ds(i, 16)] = jnp.zeros((16,), jnp.float32)

        pltpu.sync_copy(zbuf, num_sh.at[pl.ds(s * bins_per_tile, bins_per_tile)])
        pltpu.sync_copy(zbuf, den_sh.at[pl.ds(s * bins_per_tile, bins_per_tile)])

        pltpu.sync_copy(x_hbm, x_v)
        plsc.subcore_barrier()

        nch = lax.div(total_chunks - 1 - s, _NS) + 1

        def process(adj_hbm, is_neg):
            def base(kk):
                return (s + kk * _NS) * _W

            def fire_in(kk, m):
                b = base(kk)
                pltpu.async_copy(adj_hbm.at[0, pl.ds(b, _W)],
                                 bufs[m][0], sem_in)
                pltpu.async_copy(adj_hbm.at[1, pl.ds(b, _W)],
                                 bufs[m][1], sem_in)

            def wait_in(kk, m):
                b = base(kk)
                pltpu.make_async_copy(adj_hbm.at[0, pl.ds(b, _W)],
                                      bufs[m][0], sem_in).wait()
                pltpu.make_async_copy(adj_hbm.at[1, pl.ds(b, _W)],
                                      bufs[m][1], sem_in).wait()

            def drain_scatters(m):
                ci, _, nb, eb = bufs[m]
                pltpu.make_async_copy(nb, num_sh.at[ci], sem).wait()
                pltpu.make_async_copy(eb, den_sh.at[ci], sem).wait()

            def compute_and_fire(m, kk):
                ci, vi_b, nb, eb = bufs[m]

                @plsc.parallel_loop(0, _W, step=16, unroll=8)
                def _(i):
                    vi = vi_b[pl.ds(i, 16)]
                    xg = plsc.load_gather(x_v, [vi])
                    lit = (1.0 - xg) if is_neg else xg
                    e = jnp.exp(lit * _P)
                    nb[pl.ds(i, 16)] = lit * e
                    eb[pl.ds(i, 16)] = e

                pltpu.async_copy(nb, num_sh.at[ci], sem, add=True)
                pltpu.async_copy(eb, den_sh.at[ci], sem, add=True)

            fire_in(0, 0)

            @pl.loop(0, nch + 2, step=3)
            def _(kk):
                for p in range(3):
                    m, m1 = p, (p + 1) % 3
                    j = kk + p

                    @pl.when((j >= 2) & (j - 2 < nch))
                    def _():
                        drain_scatters(m1)

                    @pl.when(j + 1 < nch)
                    def _():
                        fire_in(j + 1, m1)

                    @pl.when(j < nch)
                    def _():
                        wait_in(j, m)
                        compute_and_fire(m, j)

        @pl.when(c == 0)
        def _():
            process(pos_hbm, False)

        @pl.when(c == 1)
        def _():
            process(neg_hbm, True)

        plsc.subcore_barrier()
        base = c * n_pad + s * bins_per_tile
        pltpu.sync_copy(num_sh.at[pl.ds(s * bins_per_tile, bins_per_tile)], zbuf)
        pltpu.sync_copy(zbuf, num_out.at[pl.ds(base, bins_per_tile)])
        pltpu.sync_copy(den_sh.at[pl.ds(s * bins_per_tile, bins_per_tile)], zbuf)
        pltpu.sync_copy(zbuf, den_out.at[pl.ds(base, bins_per_tile)])

    return k(x, pos, neg)


def _tc_loss(num_flat, den_flat, n_pad, num_clauses):
    rows = n_pad // 128

    def body(n_ref, d_ref, o_ref):
        n = (n_ref[pl.ds(0, n_pad)] + n_ref[pl.ds(n_pad, n_pad)]).reshape(
            rows, 128)
        d = (d_ref[pl.ds(0, n_pad)] + d_ref[pl.ds(n_pad, n_pad)]).reshape(
            rows, 128)
        r = n / d
        sm = 1.0 / (1.0 + jnp.exp(_A * (0.5 - r)))
        idx = (lax.broadcasted_iota(jnp.int32, (rows, 128), 0) * 128
               + lax.broadcasted_iota(jnp.int32, (rows, 128), 1))
        term = jnp.where(idx < num_clauses, jnp.log(sm), 0.0)
        o_ref[0, 0] = -jnp.sum(term)

    out = pl.pallas_call(
        body,
        out_shape=jax.ShapeDtypeStruct((1, 1), jnp.float32),
        out_specs=pl.BlockSpec(memory_space=pltpu.SMEM),
    )(num_flat, den_flat)
    return out[0, 0]


def kernel(xv, adj_pos, adj_neg):
    x = xv.reshape(-1)
    v_nodes = x.shape[0]
    num_clauses = v_nodes
    e_edges = adj_pos.shape[1]
    assert adj_neg.shape[1] == e_edges
    assert e_edges % _W == 0

    n_pad = ((num_clauses + _NS * 16 - 1) // (_NS * 16)) * (_NS * 16)
    if n_pad == num_clauses:
        n_pad += _NS * 16

    num_flat, den_flat = _sc_segment_sums(x, adj_pos, adj_neg, n_pad)
    return _tc_loss(num_flat, den_flat, n_pad, num_clauses)
